# baseline (speedup 1.0000x reference)
import functools

import jax
import jax.numpy as jnp
from jax import lax
from jax.experimental import pallas as pl
from jax.experimental.pallas import tpu as pltpu
from jax.experimental.pallas import tpu_sc as plsc

D = 64
NC = 2
NS = 16
NW = NC * NS
B = 4096 * 200
ROWS_PER_W = B // NW
IDXW = 128
IDX_ROWS_PER_W = ROWS_PER_W // IDXW
CHUNK_IDX_ROWS = 4
CHUNK = CHUNK_IDX_ROWS * IDXW
N_CHUNKS = ROWS_PER_W // CHUNK
GROUPS = CHUNK // 16


def _emb_body(table_hbm, idx_hbm, out_hbm, idx_v, rows_v, gsem):
    wid = lax.axis_index("s") * NC + lax.axis_index("c")
    pltpu.sync_copy(idx_hbm.at[wid], idx_v)

    def chunk_body(c, carry):
        copies = [
            pltpu.async_copy(
                table_hbm.at[idx_v.at[c * CHUNK_IDX_ROWS + j]],
                rows_v.at[pl.ds(j * IDXW, IDXW)],
                gsem,
            )
            for j in range(CHUNK_IDX_ROWS)
        ]
        for cp in copies:
            cp.wait()

        def group_body(g, carry2):
            row = c * CHUNK_IDX_ROWS + g // (IDXW // 16)
            off = (g % (IDXW // 16)) * 16
            v = idx_v[row, pl.ds(off, 16)]
            m = v == 0
            nz = jnp.max(m.astype(jnp.int32))

            @pl.when(nz > 0)
            def _():
                rows0 = g * 16 + lax.iota(jnp.int32, 16)
                zeros = jnp.zeros((16,), jnp.float32)

                def col_body(col, carry3):
                    cols = jnp.full((16,), col, jnp.int32)
                    plsc.store_scatter(rows_v, [rows0, cols], zeros, mask=m)
                    return carry3

                lax.fori_loop(0, D, col_body, 0)

            return carry2

        lax.fori_loop(0, GROUPS, group_body, 0)

        base = wid * ROWS_PER_W + c * CHUNK
        pltpu.sync_copy(rows_v, out_hbm.at[pl.ds(base, CHUNK)])
        return carry

    lax.fori_loop(0, N_CHUNKS, chunk_body, 0)


_emb = functools.partial(
    pl.kernel,
    mesh=plsc.VectorSubcoreMesh(core_axis_name="c", subcore_axis_name="s"),
    compiler_params=pltpu.CompilerParams(
        use_tc_tiling_on_sc=False, needs_layout_passes=False
    ),
    out_type=jax.ShapeDtypeStruct((B, D), jnp.float32),
    scratch_types=[
        pltpu.VMEM((IDX_ROWS_PER_W, IDXW), jnp.int32),
        pltpu.VMEM((CHUNK, D), jnp.float32),
        pltpu.SemaphoreType.DMA,
    ],
)(_emb_body)


def kernel(input_batch, table):
    bsz, seq = input_batch.shape
    idx = input_batch.reshape(-1).astype(jnp.int32)
    idx = idx.reshape(NW, IDX_ROWS_PER_W, IDXW)
    out = _emb(table, idx)
    return out.reshape(bsz, seq, D)
